# baseline (speedup 1.0000x reference)
import functools

import jax
import jax.numpy as jnp
from jax import lax
from jax.experimental import pallas as pl
from jax.experimental.pallas import tpu as pltpu
from jax.experimental.pallas import tpu_sc as plsc

N_E = 1024
D = 256
N_ROWS = 64 * 256
BLK = 512
GRID = N_ROWS // BLK


def _vq_tc_body(x_ref, embt_ref, idx_ref, loss_ref, ppl_ref,
                ent_ref, e2_ref, npl_ref, esq_ref, cnt_ref, lacc_ref):
    step = pl.program_id(0)

    @pl.when(step == 0)
    def _init():
        embt = embt_ref[...]
        esq = jnp.sum(embt * embt, axis=0, keepdims=True)
        npl = jnp.sqrt(esq) + 1e-4
        ent = embt / npl
        ent_ref[...] = ent
        e2_ref[...] = jnp.sum(ent * ent, axis=0, keepdims=True)
        npl_ref[...] = npl
        esq_ref[...] = esq
        cnt_ref[...] = jnp.zeros_like(cnt_ref)
        lacc_ref[...] = jnp.zeros_like(lacc_ref)

    x = x_ref[...]
    s = lax.dot_general(x, ent_ref[...], (((1,), (0,)), ((), ())),
                        preferred_element_type=jnp.float32,
                        precision=lax.Precision.DEFAULT)
    x2 = jnp.sum(x * x, axis=1, keepdims=True)
    d = (e2_ref[...] + x2) - 2.0 * s
    m = jnp.min(d, axis=1, keepdims=True)
    iota = lax.broadcasted_iota(jnp.int32, (BLK, N_E), 1)
    idx = jnp.min(jnp.where(d == m, iota, N_E), axis=1)
    idx_ref[...] = idx.reshape(1, 1, BLK)

    onehot = (iota == idx[:, None]).astype(jnp.float32)
    cnt_ref[...] += jnp.sum(onehot, axis=0, keepdims=True)
    s_sel = jnp.sum(s * onehot, axis=1)
    np_sel = jnp.sum(npl_ref[...] * onehot, axis=1)
    es_sel = jnp.sum(esq_ref[...] * onehot, axis=1)
    part = jnp.sum(x2[:, 0] - 2.0 * (s_sel * np_sel) + es_sel)
    lacc_ref[...] += part.reshape(1, 1)

    @pl.when(step == GRID - 1)
    def _fin():
        loss_ref[...] = lacc_ref[...] * (1.0 / (N_ROWS * D))
        p = cnt_ref[...] * (1.0 / N_ROWS)
        ppl_ref[...] = jnp.exp(-jnp.sum(p * jnp.log(p + 1e-10))).reshape(1, 1)


def _vq_tc(x_flat, embt):
    return pl.pallas_call(
        _vq_tc_body,
        grid=(GRID,),
        in_specs=[
            pl.BlockSpec((BLK, D), lambda i: (i, 0)),
            pl.BlockSpec((D, N_E), lambda i: (0, 0)),
        ],
        out_specs=[
            pl.BlockSpec((1, 1, BLK), lambda i: (i, 0, 0)),
            pl.BlockSpec((1, 1), lambda i: (0, 0)),
            pl.BlockSpec((1, 1), lambda i: (0, 0)),
        ],
        out_shape=[
            jax.ShapeDtypeStruct((GRID, 1, BLK), jnp.int32),
            jax.ShapeDtypeStruct((1, 1), jnp.float32),
            jax.ShapeDtypeStruct((1, 1), jnp.float32),
        ],
        scratch_shapes=[
            pltpu.VMEM((D, N_E), jnp.float32),
            pltpu.VMEM((1, N_E), jnp.float32),
            pltpu.VMEM((1, N_E), jnp.float32),
            pltpu.VMEM((1, N_E), jnp.float32),
            pltpu.VMEM((1, N_E), jnp.float32),
            pltpu.VMEM((1, 1), jnp.float32),
        ],
    )(x_flat, embt)


_NC = 2
_NS = 16
_NW = _NC * _NS
_BPW = N_ROWS // _NW
_CH = 128
_NCH = _BPW // _CH


@functools.cache
def _make_sc_gather():
    @functools.partial(
        pl.kernel,
        mesh=plsc.VectorSubcoreMesh(core_axis_name="c", subcore_axis_name="s"),
        out_type=jax.ShapeDtypeStruct((N_ROWS, D), jnp.float32),
        scratch_types=[
            pltpu.VMEM((_NCH, _CH), jnp.int32),
            pltpu.VMEM((_CH, D), jnp.float32),
            pltpu.VMEM((_CH, D), jnp.float32),
            pltpu.SemaphoreType.DMA,
            pltpu.SemaphoreType.DMA,
            pltpu.SemaphoreType.DMA,
            pltpu.SemaphoreType.DMA,
        ],
    )
    def _sc_gather(emb_hbm, idx_hbm, out_hbm, idx_v, rows0, rows1, g0, g1, s0, s1):
        wid = lax.axis_index("s") * _NC + lax.axis_index("c")
        base = wid * _BPW
        rows = (rows0, rows1)
        gsem = (g0, g1)
        ssem = (s0, s1)

        for j in range(_NCH):
            pltpu.sync_copy(idx_hbm.at[pl.ds(base + j * _CH, _CH)], idx_v.at[j])
        pltpu.async_copy(emb_hbm.at[idx_v.at[0]], rows[0], gsem[0])
        for j in range(_NCH):
            b = j % 2
            pltpu.make_async_copy(emb_hbm.at[idx_v.at[j]], rows[b], gsem[b]).wait()
            if j + 1 < _NCH:
                nb = (j + 1) % 2
                if j - 1 >= 0:
                    pltpu.make_async_copy(
                        rows[nb], out_hbm.at[pl.ds(base + (j - 1) * _CH, _CH)],
                        ssem[nb]).wait()
                pltpu.async_copy(emb_hbm.at[idx_v.at[j + 1]], rows[nb], gsem[nb])
            pltpu.async_copy(rows[b], out_hbm.at[pl.ds(base + j * _CH, _CH)],
                             ssem[b])
        for j in (_NCH - 2, _NCH - 1):
            b = j % 2
            pltpu.make_async_copy(rows[b], out_hbm.at[pl.ds(base + j * _CH, _CH)],
                                  ssem[b]).wait()

    return _sc_gather


def kernel(x, embedding):
    x_flat = x.reshape(-1, D)
    idx_blocks, loss, ppl = _vq_tc(x_flat, embedding.T)
    indices = idx_blocks.reshape(N_ROWS)
    quantized = _make_sc_gather()(embedding, indices).reshape(x.shape)
    return (quantized, loss[0, 0], ppl[0, 0])


# speedup vs baseline: 1.2020x; 1.2020x over previous
import functools

import jax
import jax.numpy as jnp
from jax import lax
from jax.experimental import pallas as pl
from jax.experimental.pallas import tpu as pltpu
from jax.experimental.pallas import tpu_sc as plsc

N_E = 1024
D = 256
N_ROWS = 64 * 256
BLK = 512
GRID = N_ROWS // BLK


def _vq_tc_body(x_ref, embt_ref, idx_ref, loss_ref, ppl_ref,
                ent_ref, e2_ref, npl_ref, esq_ref, cnt_ref, lacc_ref):
    step = pl.program_id(0)

    @pl.when(step == 0)
    def _init():
        embt = embt_ref[...]
        esq = jnp.sum(embt * embt, axis=0, keepdims=True)
        npl = jnp.sqrt(esq) + 1e-4
        ent = embt / npl
        ent_ref[...] = ent
        e2_ref[...] = jnp.sum(ent * ent, axis=0, keepdims=True)
        npl_ref[...] = 2.0 * npl
        esq_ref[...] = esq
        cnt_ref[...] = jnp.zeros_like(cnt_ref)
        lacc_ref[...] = jnp.zeros_like(lacc_ref)

    x = x_ref[...]
    s = lax.dot_general(x, ent_ref[...], (((1,), (0,)), ((), ())),
                        preferred_element_type=jnp.float32,
                        precision=lax.Precision.DEFAULT)
    x2 = jnp.sum(x * x, axis=1, keepdims=True)
    d = (e2_ref[...] + x2) - 2.0 * s
    m = jnp.min(d, axis=1, keepdims=True)
    mask = d == m
    iota = lax.broadcasted_iota(jnp.int32, (BLK, N_E), 1)
    idx = jnp.min(jnp.where(mask, iota, N_E), axis=1, keepdims=True)
    idx_ref[...] = idx

    cnt_ref[...] += jnp.sum(jnp.where(mask, 1.0, 0.0), axis=0, keepdims=True)
    t = esq_ref[...] - npl_ref[...] * s
    part = jnp.sum(x2) + jnp.sum(jnp.where(mask, t, 0.0))
    lacc_ref[...] += part.reshape(1, 1)

    @pl.when(step == GRID - 1)
    def _fin():
        loss_ref[...] = lacc_ref[...] * (1.0 / (N_ROWS * D))
        p = cnt_ref[...] * (1.0 / N_ROWS)
        ppl_ref[...] = jnp.exp(-jnp.sum(p * jnp.log(p + 1e-10))).reshape(1, 1)


def _vq_tc(x_flat, embt):
    return pl.pallas_call(
        _vq_tc_body,
        grid=(GRID,),
        in_specs=[
            pl.BlockSpec((BLK, D), lambda i: (i, 0)),
            pl.BlockSpec((D, N_E), lambda i: (0, 0)),
        ],
        out_specs=[
            pl.BlockSpec((BLK, 1), lambda i: (i, 0)),
            pl.BlockSpec((1, 1), lambda i: (0, 0)),
            pl.BlockSpec((1, 1), lambda i: (0, 0)),
        ],
        out_shape=[
            jax.ShapeDtypeStruct((N_ROWS, 1), jnp.int32),
            jax.ShapeDtypeStruct((1, 1), jnp.float32),
            jax.ShapeDtypeStruct((1, 1), jnp.float32),
        ],
        scratch_shapes=[
            pltpu.VMEM((D, N_E), jnp.float32),
            pltpu.VMEM((1, N_E), jnp.float32),
            pltpu.VMEM((1, N_E), jnp.float32),
            pltpu.VMEM((1, N_E), jnp.float32),
            pltpu.VMEM((1, N_E), jnp.float32),
            pltpu.VMEM((1, 1), jnp.float32),
        ],
    )(x_flat, embt)


_NC = 2
_NS = 16
_NW = _NC * _NS
_BPW = N_ROWS // _NW
_CH = 128
_NCH = _BPW // _CH


@functools.cache
def _make_sc_gather():
    @functools.partial(
        pl.kernel,
        mesh=plsc.VectorSubcoreMesh(core_axis_name="c", subcore_axis_name="s"),
        out_type=jax.ShapeDtypeStruct((N_ROWS, D), jnp.float32),
        scratch_types=[
            pltpu.VMEM((_NCH, _CH), jnp.int32),
            pltpu.VMEM((_CH, D), jnp.float32),
            pltpu.VMEM((_CH, D), jnp.float32),
            pltpu.SemaphoreType.DMA,
            pltpu.SemaphoreType.DMA,
            pltpu.SemaphoreType.DMA,
            pltpu.SemaphoreType.DMA,
        ],
    )
    def _sc_gather(emb_hbm, idx_hbm, out_hbm, idx_v, rows0, rows1, g0, g1, s0, s1):
        wid = lax.axis_index("s") * _NC + lax.axis_index("c")
        base = wid * _BPW
        rows = (rows0, rows1)
        gsem = (g0, g1)
        ssem = (s0, s1)

        for j in range(_NCH):
            pltpu.sync_copy(idx_hbm.at[pl.ds(base + j * _CH, _CH)], idx_v.at[j])
        pltpu.async_copy(emb_hbm.at[idx_v.at[0]], rows[0], gsem[0])
        for j in range(_NCH):
            b = j % 2
            pltpu.make_async_copy(emb_hbm.at[idx_v.at[j]], rows[b], gsem[b]).wait()
            if j + 1 < _NCH:
                nb = (j + 1) % 2
                if j - 1 >= 0:
                    pltpu.make_async_copy(
                        rows[nb], out_hbm.at[pl.ds(base + (j - 1) * _CH, _CH)],
                        ssem[nb]).wait()
                pltpu.async_copy(emb_hbm.at[idx_v.at[j + 1]], rows[nb], gsem[nb])
            pltpu.async_copy(rows[b], out_hbm.at[pl.ds(base + j * _CH, _CH)],
                             ssem[b])
        for j in (_NCH - 2, _NCH - 1):
            b = j % 2
            pltpu.make_async_copy(rows[b], out_hbm.at[pl.ds(base + j * _CH, _CH)],
                                  ssem[b]).wait()

    return _sc_gather


def kernel(x, embedding):
    x_flat = x.reshape(-1, D)
    idx_blocks, loss, ppl = _vq_tc(x_flat, embedding.T)
    indices = idx_blocks.reshape(N_ROWS)
    quantized = _make_sc_gather()(embedding, indices).reshape(x.shape)
    return (quantized, loss[0, 0], ppl[0, 0])


# speedup vs baseline: 1.2957x; 1.0779x over previous
import functools

import jax
import jax.numpy as jnp
from jax import lax
from jax.experimental import pallas as pl
from jax.experimental.pallas import tpu as pltpu
from jax.experimental.pallas import tpu_sc as plsc

N_E = 1024
D = 256
N_ROWS = 64 * 256
BLK = 1024
GRID = N_ROWS // BLK


def _vq_tc_body(x_ref, embt_ref, idx_ref, loss_ref, ppl_ref,
                ent_ref, e2_ref, npl_ref, esq_ref, cnt_ref, lacc_ref):
    step = pl.program_id(0)

    @pl.when(step == 0)
    def _init():
        embt = embt_ref[...]
        esq = jnp.sum(embt * embt, axis=0, keepdims=True)
        npl = jnp.sqrt(esq) + 1e-4
        ent = embt / npl
        ent_ref[...] = ent
        e2_ref[...] = jnp.sum(ent * ent, axis=0, keepdims=True)
        npl_ref[...] = 2.0 * npl
        esq_ref[...] = esq
        cnt_ref[...] = jnp.zeros_like(cnt_ref)
        lacc_ref[...] = jnp.zeros_like(lacc_ref)

    x = x_ref[...]
    s = lax.dot_general(x, ent_ref[...], (((1,), (0,)), ((), ())),
                        preferred_element_type=jnp.float32,
                        precision=lax.Precision.DEFAULT)
    x2 = jnp.sum(x * x, axis=1, keepdims=True)
    d = (e2_ref[...] + x2) - 2.0 * s
    m = jnp.min(d, axis=1, keepdims=True)
    mask = d == m
    iota = lax.broadcasted_iota(jnp.int32, (BLK, N_E), 1)
    idx = jnp.min(jnp.where(mask, iota, N_E), axis=1, keepdims=True)
    idx_ref[...] = idx

    cnt_ref[...] += jnp.sum(jnp.where(mask, 1.0, 0.0), axis=0, keepdims=True)
    t = esq_ref[...] - npl_ref[...] * s
    part = jnp.sum(x2) + jnp.sum(jnp.where(mask, t, 0.0))
    lacc_ref[...] += part.reshape(1, 1)

    @pl.when(step == GRID - 1)
    def _fin():
        loss_ref[...] = lacc_ref[...] * (1.0 / (N_ROWS * D))
        p = cnt_ref[...] * (1.0 / N_ROWS)
        ppl_ref[...] = jnp.exp(-jnp.sum(p * jnp.log(p + 1e-10))).reshape(1, 1)


def _vq_tc(x_flat, embt):
    return pl.pallas_call(
        _vq_tc_body,
        grid=(GRID,),
        in_specs=[
            pl.BlockSpec((BLK, D), lambda i: (i, 0)),
            pl.BlockSpec((D, N_E), lambda i: (0, 0)),
        ],
        out_specs=[
            pl.BlockSpec((BLK, 1), lambda i: (i, 0)),
            pl.BlockSpec((1, 1), lambda i: (0, 0)),
            pl.BlockSpec((1, 1), lambda i: (0, 0)),
        ],
        out_shape=[
            jax.ShapeDtypeStruct((N_ROWS, 1), jnp.int32),
            jax.ShapeDtypeStruct((1, 1), jnp.float32),
            jax.ShapeDtypeStruct((1, 1), jnp.float32),
        ],
        scratch_shapes=[
            pltpu.VMEM((D, N_E), jnp.float32),
            pltpu.VMEM((1, N_E), jnp.float32),
            pltpu.VMEM((1, N_E), jnp.float32),
            pltpu.VMEM((1, N_E), jnp.float32),
            pltpu.VMEM((1, N_E), jnp.float32),
            pltpu.VMEM((1, 1), jnp.float32),
        ],
    )(x_flat, embt)


_NC = 2
_NS = 16
_NW = _NC * _NS
_BPW = N_ROWS // _NW
_CH = 128
_NCH = _BPW // _CH


@functools.cache
def _make_sc_gather():
    @functools.partial(
        pl.kernel,
        mesh=plsc.VectorSubcoreMesh(core_axis_name="c", subcore_axis_name="s"),
        out_type=jax.ShapeDtypeStruct((N_ROWS, D), jnp.float32),
        scratch_types=[
            pltpu.VMEM((_NCH, _CH), jnp.int32),
            pltpu.VMEM((_CH, D), jnp.float32),
            pltpu.VMEM((_CH, D), jnp.float32),
            pltpu.SemaphoreType.DMA,
            pltpu.SemaphoreType.DMA,
            pltpu.SemaphoreType.DMA,
            pltpu.SemaphoreType.DMA,
        ],
    )
    def _sc_gather(emb_hbm, idx_hbm, out_hbm, idx_v, rows0, rows1, g0, g1, s0, s1):
        wid = lax.axis_index("s") * _NC + lax.axis_index("c")
        base = wid * _BPW
        rows = (rows0, rows1)
        gsem = (g0, g1)
        ssem = (s0, s1)

        for j in range(_NCH):
            pltpu.sync_copy(idx_hbm.at[pl.ds(base + j * _CH, _CH)], idx_v.at[j])
        pltpu.async_copy(emb_hbm.at[idx_v.at[0]], rows[0], gsem[0])
        for j in range(_NCH):
            b = j % 2
            pltpu.make_async_copy(emb_hbm.at[idx_v.at[j]], rows[b], gsem[b]).wait()
            if j + 1 < _NCH:
                nb = (j + 1) % 2
                if j - 1 >= 0:
                    pltpu.make_async_copy(
                        rows[nb], out_hbm.at[pl.ds(base + (j - 1) * _CH, _CH)],
                        ssem[nb]).wait()
                pltpu.async_copy(emb_hbm.at[idx_v.at[j + 1]], rows[nb], gsem[nb])
            pltpu.async_copy(rows[b], out_hbm.at[pl.ds(base + j * _CH, _CH)],
                             ssem[b])
        for j in (_NCH - 2, _NCH - 1):
            b = j % 2
            pltpu.make_async_copy(rows[b], out_hbm.at[pl.ds(base + j * _CH, _CH)],
                                  ssem[b]).wait()

    return _sc_gather


def kernel(x, embedding):
    x_flat = x.reshape(-1, D)
    idx_blocks, loss, ppl = _vq_tc(x_flat, embedding.T)
    indices = idx_blocks.reshape(N_ROWS)
    quantized = _make_sc_gather()(embedding, indices).reshape(x.shape)
    return (quantized, loss[0, 0], ppl[0, 0])


# speedup vs baseline: 1.3226x; 1.0207x over previous
import functools

import jax
import jax.numpy as jnp
from jax import lax
from jax.experimental import pallas as pl
from jax.experimental.pallas import tpu as pltpu
from jax.experimental.pallas import tpu_sc as plsc

N_E = 1024
D = 256
N_ROWS = 64 * 256
BLK = 2048
GRID = N_ROWS // BLK


def _vq_tc_body(x_ref, embt_ref, idx_ref, loss_ref, ppl_ref,
                ent_ref, e2_ref, npl_ref, esq_ref, cnt_ref, lacc_ref):
    step = pl.program_id(0)

    @pl.when(step == 0)
    def _init():
        embt = embt_ref[...]
        esq = jnp.sum(embt * embt, axis=0, keepdims=True)
        npl = jnp.sqrt(esq) + 1e-4
        ent = embt / npl
        ent_ref[...] = ent
        e2_ref[...] = jnp.sum(ent * ent, axis=0, keepdims=True)
        npl_ref[...] = 2.0 * npl
        esq_ref[...] = esq
        cnt_ref[...] = jnp.zeros_like(cnt_ref)
        lacc_ref[...] = jnp.zeros_like(lacc_ref)

    x = x_ref[...]
    s = lax.dot_general(x, ent_ref[...], (((1,), (0,)), ((), ())),
                        preferred_element_type=jnp.float32,
                        precision=lax.Precision.DEFAULT)
    x2 = jnp.sum(x * x, axis=1, keepdims=True)
    d = (e2_ref[...] + x2) - 2.0 * s
    m = jnp.min(d, axis=1, keepdims=True)
    mask = d == m
    iota = lax.broadcasted_iota(jnp.int32, (BLK, N_E), 1)
    idx = jnp.min(jnp.where(mask, iota, N_E), axis=1, keepdims=True)
    idx_ref[...] = idx

    cnt_ref[...] += jnp.sum(jnp.where(mask, 1.0, 0.0), axis=0, keepdims=True)
    t = esq_ref[...] - npl_ref[...] * s
    part = jnp.sum(x2) + jnp.sum(jnp.where(mask, t, 0.0))
    lacc_ref[...] += part.reshape(1, 1)

    @pl.when(step == GRID - 1)
    def _fin():
        loss_ref[...] = lacc_ref[...] * (1.0 / (N_ROWS * D))
        p = cnt_ref[...] * (1.0 / N_ROWS)
        ppl_ref[...] = jnp.exp(-jnp.sum(p * jnp.log(p + 1e-10))).reshape(1, 1)


def _vq_tc(x_flat, embt):
    return pl.pallas_call(
        _vq_tc_body,
        grid=(GRID,),
        in_specs=[
            pl.BlockSpec((BLK, D), lambda i: (i, 0)),
            pl.BlockSpec((D, N_E), lambda i: (0, 0)),
        ],
        out_specs=[
            pl.BlockSpec((BLK, 1), lambda i: (i, 0)),
            pl.BlockSpec((1, 1), lambda i: (0, 0)),
            pl.BlockSpec((1, 1), lambda i: (0, 0)),
        ],
        out_shape=[
            jax.ShapeDtypeStruct((N_ROWS, 1), jnp.int32),
            jax.ShapeDtypeStruct((1, 1), jnp.float32),
            jax.ShapeDtypeStruct((1, 1), jnp.float32),
        ],
        scratch_shapes=[
            pltpu.VMEM((D, N_E), jnp.float32),
            pltpu.VMEM((1, N_E), jnp.float32),
            pltpu.VMEM((1, N_E), jnp.float32),
            pltpu.VMEM((1, N_E), jnp.float32),
            pltpu.VMEM((1, N_E), jnp.float32),
            pltpu.VMEM((1, 1), jnp.float32),
        ],
    )(x_flat, embt)


_NC = 2
_NS = 16
_NW = _NC * _NS
_BPW = N_ROWS // _NW
_CH = 128
_NCH = _BPW // _CH


@functools.cache
def _make_sc_gather():
    @functools.partial(
        pl.kernel,
        mesh=plsc.VectorSubcoreMesh(core_axis_name="c", subcore_axis_name="s"),
        out_type=jax.ShapeDtypeStruct((N_ROWS, D), jnp.float32),
        scratch_types=[
            pltpu.VMEM((_NCH, _CH), jnp.int32),
            pltpu.VMEM((_CH, D), jnp.float32),
            pltpu.VMEM((_CH, D), jnp.float32),
            pltpu.SemaphoreType.DMA,
            pltpu.SemaphoreType.DMA,
            pltpu.SemaphoreType.DMA,
            pltpu.SemaphoreType.DMA,
        ],
    )
    def _sc_gather(emb_hbm, idx_hbm, out_hbm, idx_v, rows0, rows1, g0, g1, s0, s1):
        wid = lax.axis_index("s") * _NC + lax.axis_index("c")
        base = wid * _BPW
        rows = (rows0, rows1)
        gsem = (g0, g1)
        ssem = (s0, s1)

        for j in range(_NCH):
            pltpu.sync_copy(idx_hbm.at[pl.ds(base + j * _CH, _CH)], idx_v.at[j])
        pltpu.async_copy(emb_hbm.at[idx_v.at[0]], rows[0], gsem[0])
        for j in range(_NCH):
            b = j % 2
            pltpu.make_async_copy(emb_hbm.at[idx_v.at[j]], rows[b], gsem[b]).wait()
            if j + 1 < _NCH:
                nb = (j + 1) % 2
                if j - 1 >= 0:
                    pltpu.make_async_copy(
                        rows[nb], out_hbm.at[pl.ds(base + (j - 1) * _CH, _CH)],
                        ssem[nb]).wait()
                pltpu.async_copy(emb_hbm.at[idx_v.at[j + 1]], rows[nb], gsem[nb])
            pltpu.async_copy(rows[b], out_hbm.at[pl.ds(base + j * _CH, _CH)],
                             ssem[b])
        for j in (_NCH - 2, _NCH - 1):
            b = j % 2
            pltpu.make_async_copy(rows[b], out_hbm.at[pl.ds(base + j * _CH, _CH)],
                                  ssem[b]).wait()

    return _sc_gather


def kernel(x, embedding):
    x_flat = x.reshape(-1, D)
    idx_blocks, loss, ppl = _vq_tc(x_flat, embedding.T)
    indices = idx_blocks.reshape(N_ROWS)
    quantized = _make_sc_gather()(embedding, indices).reshape(x.shape)
    return (quantized, loss[0, 0], ppl[0, 0])


# speedup vs baseline: 1.3595x; 1.0279x over previous
import functools

import jax
import jax.numpy as jnp
from jax import lax
from jax.experimental import pallas as pl
from jax.experimental.pallas import tpu as pltpu
from jax.experimental.pallas import tpu_sc as plsc

N_E = 1024
D = 256
N_ROWS = 64 * 256
BLK = 2048
GRID = N_ROWS // BLK


def _vq_tc_body(x_ref, emb_ref, idx_ref, loss_ref, ppl_ref,
                ent_ref, e2_ref, npl_ref, esq_ref, cnt_ref, lacc_ref):
    step = pl.program_id(0)

    @pl.when(step == 0)
    def _init():
        embt = emb_ref[...].T
        esq = jnp.sum(embt * embt, axis=0, keepdims=True)
        npl = jnp.sqrt(esq) + 1e-4
        ent = embt / npl
        ent_ref[...] = ent
        e2_ref[...] = jnp.sum(ent * ent, axis=0, keepdims=True)
        npl_ref[...] = 2.0 * npl
        esq_ref[...] = esq
        cnt_ref[...] = jnp.zeros_like(cnt_ref)
        lacc_ref[...] = jnp.zeros_like(lacc_ref)

    x = x_ref[...]
    s = lax.dot_general(x, ent_ref[...], (((1,), (0,)), ((), ())),
                        preferred_element_type=jnp.float32,
                        precision=lax.Precision.DEFAULT)
    x2 = jnp.sum(x * x, axis=1, keepdims=True)
    d = (e2_ref[...] + x2) - 2.0 * s
    m = jnp.min(d, axis=1, keepdims=True)
    mask = d == m
    iota = lax.broadcasted_iota(jnp.int32, (BLK, N_E), 1)
    idx = jnp.min(jnp.where(mask, iota, N_E), axis=1, keepdims=True)
    idx_ref[...] = idx

    cnt_ref[...] += jnp.sum(jnp.where(mask, 1.0, 0.0), axis=0, keepdims=True)
    t = esq_ref[...] - npl_ref[...] * s
    part = jnp.sum(x2) + jnp.sum(jnp.where(mask, t, 0.0))
    lacc_ref[...] += part.reshape(1, 1)

    @pl.when(step == GRID - 1)
    def _fin():
        loss_ref[...] = lacc_ref[...] * (1.0 / (N_ROWS * D))
        p = cnt_ref[...] * (1.0 / N_ROWS)
        ppl_ref[...] = jnp.exp(-jnp.sum(p * jnp.log(p + 1e-10))).reshape(1, 1)


def _vq_tc(x_flat, emb):
    return pl.pallas_call(
        _vq_tc_body,
        grid=(GRID,),
        in_specs=[
            pl.BlockSpec((BLK, D), lambda i: (i, 0)),
            pl.BlockSpec((N_E, D), lambda i: (0, 0)),
        ],
        out_specs=[
            pl.BlockSpec((BLK, 1), lambda i: (i, 0)),
            pl.BlockSpec((1, 1), lambda i: (0, 0)),
            pl.BlockSpec((1, 1), lambda i: (0, 0)),
        ],
        out_shape=[
            jax.ShapeDtypeStruct((N_ROWS, 1), jnp.int32),
            jax.ShapeDtypeStruct((1, 1), jnp.float32),
            jax.ShapeDtypeStruct((1, 1), jnp.float32),
        ],
        scratch_shapes=[
            pltpu.VMEM((D, N_E), jnp.float32),
            pltpu.VMEM((1, N_E), jnp.float32),
            pltpu.VMEM((1, N_E), jnp.float32),
            pltpu.VMEM((1, N_E), jnp.float32),
            pltpu.VMEM((1, N_E), jnp.float32),
            pltpu.VMEM((1, 1), jnp.float32),
        ],
    )(x_flat, emb)


_NC = 2
_NS = 16
_NW = _NC * _NS
_BPW = N_ROWS // _NW
_CH = 128
_NCH = _BPW // _CH


@functools.cache
def _make_sc_gather():
    @functools.partial(
        pl.kernel,
        mesh=plsc.VectorSubcoreMesh(core_axis_name="c", subcore_axis_name="s"),
        out_type=jax.ShapeDtypeStruct((N_ROWS, D), jnp.float32),
        scratch_types=[
            pltpu.VMEM((_NCH, _CH), jnp.int32),
            pltpu.VMEM((_CH, D), jnp.float32),
            pltpu.VMEM((_CH, D), jnp.float32),
            pltpu.SemaphoreType.DMA,
            pltpu.SemaphoreType.DMA,
            pltpu.SemaphoreType.DMA,
            pltpu.SemaphoreType.DMA,
        ],
    )
    def _sc_gather(emb_hbm, idx_hbm, out_hbm, idx_v, rows0, rows1, g0, g1, s0, s1):
        wid = lax.axis_index("s") * _NC + lax.axis_index("c")
        base = wid * _BPW
        rows = (rows0, rows1)
        gsem = (g0, g1)
        ssem = (s0, s1)

        for j in range(_NCH):
            pltpu.sync_copy(idx_hbm.at[pl.ds(base + j * _CH, _CH)], idx_v.at[j])
        pltpu.async_copy(emb_hbm.at[idx_v.at[0]], rows[0], gsem[0])
        for j in range(_NCH):
            b = j % 2
            pltpu.make_async_copy(emb_hbm.at[idx_v.at[j]], rows[b], gsem[b]).wait()
            if j + 1 < _NCH:
                nb = (j + 1) % 2
                if j - 1 >= 0:
                    pltpu.make_async_copy(
                        rows[nb], out_hbm.at[pl.ds(base + (j - 1) * _CH, _CH)],
                        ssem[nb]).wait()
                pltpu.async_copy(emb_hbm.at[idx_v.at[j + 1]], rows[nb], gsem[nb])
            pltpu.async_copy(rows[b], out_hbm.at[pl.ds(base + j * _CH, _CH)],
                             ssem[b])
        for j in (_NCH - 2, _NCH - 1):
            b = j % 2
            pltpu.make_async_copy(rows[b], out_hbm.at[pl.ds(base + j * _CH, _CH)],
                                  ssem[b]).wait()

    return _sc_gather


def kernel(x, embedding):
    x_flat = x.reshape(-1, D)
    idx_blocks, loss, ppl = _vq_tc(x_flat, embedding)
    indices = idx_blocks.reshape(N_ROWS)
    quantized = _make_sc_gather()(embedding, indices).reshape(x.shape)
    return (quantized, loss[0, 0], ppl[0, 0])


# speedup vs baseline: 1.3768x; 1.0128x over previous
import functools

import jax
import jax.numpy as jnp
from jax import lax
from jax.experimental import pallas as pl
from jax.experimental.pallas import tpu as pltpu
from jax.experimental.pallas import tpu_sc as plsc

N_E = 1024
D = 256
N_ROWS = 64 * 256
BLK = 4096
GRID = N_ROWS // BLK


def _vq_tc_body(x_ref, emb_ref, idx_ref, loss_ref, ppl_ref,
                ent_ref, e2_ref, npl_ref, esq_ref, cnt_ref, lacc_ref):
    step = pl.program_id(0)

    @pl.when(step == 0)
    def _init():
        embt = emb_ref[...].T
        esq = jnp.sum(embt * embt, axis=0, keepdims=True)
        npl = jnp.sqrt(esq) + 1e-4
        ent = embt / npl
        ent_ref[...] = ent
        e2_ref[...] = jnp.sum(ent * ent, axis=0, keepdims=True)
        npl_ref[...] = 2.0 * npl
        esq_ref[...] = esq
        cnt_ref[...] = jnp.zeros_like(cnt_ref)
        lacc_ref[...] = jnp.zeros_like(lacc_ref)

    x = x_ref[...]
    s = lax.dot_general(x, ent_ref[...], (((1,), (0,)), ((), ())),
                        preferred_element_type=jnp.float32,
                        precision=lax.Precision.DEFAULT)
    x2 = jnp.sum(x * x, axis=1, keepdims=True)
    d = (e2_ref[...] + x2) - 2.0 * s
    m = jnp.min(d, axis=1, keepdims=True)
    mask = d == m
    iota = lax.broadcasted_iota(jnp.int32, (BLK, N_E), 1)
    idx = jnp.min(jnp.where(mask, iota, N_E), axis=1, keepdims=True)
    idx_ref[...] = idx

    cnt_ref[...] += jnp.sum(jnp.where(mask, 1.0, 0.0), axis=0, keepdims=True)
    t = esq_ref[...] - npl_ref[...] * s
    part = jnp.sum(x2) + jnp.sum(jnp.where(mask, t, 0.0))
    lacc_ref[...] += part.reshape(1, 1)

    @pl.when(step == GRID - 1)
    def _fin():
        loss_ref[...] = lacc_ref[...] * (1.0 / (N_ROWS * D))
        p = cnt_ref[...] * (1.0 / N_ROWS)
        ppl_ref[...] = jnp.exp(-jnp.sum(p * jnp.log(p + 1e-10))).reshape(1, 1)


def _vq_tc(x_flat, emb):
    return pl.pallas_call(
        _vq_tc_body,
        grid=(GRID,),
        in_specs=[
            pl.BlockSpec((BLK, D), lambda i: (i, 0)),
            pl.BlockSpec((N_E, D), lambda i: (0, 0)),
        ],
        out_specs=[
            pl.BlockSpec((BLK, 1), lambda i: (i, 0)),
            pl.BlockSpec((1, 1), lambda i: (0, 0)),
            pl.BlockSpec((1, 1), lambda i: (0, 0)),
        ],
        out_shape=[
            jax.ShapeDtypeStruct((N_ROWS, 1), jnp.int32),
            jax.ShapeDtypeStruct((1, 1), jnp.float32),
            jax.ShapeDtypeStruct((1, 1), jnp.float32),
        ],
        scratch_shapes=[
            pltpu.VMEM((D, N_E), jnp.float32),
            pltpu.VMEM((1, N_E), jnp.float32),
            pltpu.VMEM((1, N_E), jnp.float32),
            pltpu.VMEM((1, N_E), jnp.float32),
            pltpu.VMEM((1, N_E), jnp.float32),
            pltpu.VMEM((1, 1), jnp.float32),
        ],
    )(x_flat, emb)


_NC = 2
_NS = 16
_NW = _NC * _NS
_BPW = N_ROWS // _NW
_CH = 128
_NCH = _BPW // _CH


@functools.cache
def _make_sc_gather():
    @functools.partial(
        pl.kernel,
        mesh=plsc.VectorSubcoreMesh(core_axis_name="c", subcore_axis_name="s"),
        out_type=jax.ShapeDtypeStruct((N_ROWS, D), jnp.float32),
        scratch_types=[
            pltpu.VMEM((_NCH, _CH), jnp.int32),
            pltpu.VMEM((_CH, D), jnp.float32),
            pltpu.VMEM((_CH, D), jnp.float32),
            pltpu.SemaphoreType.DMA,
            pltpu.SemaphoreType.DMA,
            pltpu.SemaphoreType.DMA,
            pltpu.SemaphoreType.DMA,
        ],
    )
    def _sc_gather(emb_hbm, idx_hbm, out_hbm, idx_v, rows0, rows1,
                   g0, g1, s0, s1):
        wid = lax.axis_index("s") * _NC + lax.axis_index("c")
        base = wid * _BPW
        rows = (rows0, rows1)
        gsem = (g0, g1)
        ssem = (s0, s1)

        for j in range(_NCH):
            pltpu.sync_copy(idx_hbm.at[pl.ds(base + j * _CH, _CH)], idx_v.at[j])
        pltpu.async_copy(emb_hbm.at[idx_v.at[0]], rows[0], gsem[0])
        for j in range(_NCH):
            b = j % 2
            pltpu.make_async_copy(emb_hbm.at[idx_v.at[j]], rows[b], gsem[b]).wait()
            if j + 1 < _NCH:
                nb = (j + 1) % 2
                if j - 1 >= 0:
                    pltpu.make_async_copy(
                        rows[nb], out_hbm.at[pl.ds(base + (j - 1) * _CH, _CH)],
                        ssem[nb]).wait()
                pltpu.async_copy(emb_hbm.at[idx_v.at[j + 1]], rows[nb], gsem[nb])
            pltpu.async_copy(rows[b], out_hbm.at[pl.ds(base + j * _CH, _CH)],
                             ssem[b])
        for j in (_NCH - 2, _NCH - 1):
            b = j % 2
            pltpu.make_async_copy(rows[b], out_hbm.at[pl.ds(base + j * _CH, _CH)],
                                  ssem[b]).wait()

    return _sc_gather


def kernel(x, embedding):
    x_flat = x.reshape(-1, D)
    idx_blocks, loss, ppl = _vq_tc(x_flat, embedding)
    indices = idx_blocks.reshape(N_ROWS)
    quantized = _make_sc_gather()(embedding, indices).reshape(x.shape)
    return (quantized, loss[0, 0], ppl[0, 0])
